# baseline (speedup 1.0000x reference)
import functools

import jax
import jax.numpy as jnp
from jax import lax
from jax.experimental import pallas as pl
from jax.experimental.pallas import tpu as pltpu
from jax.experimental.pallas import tpu_sc as plsc

B_ROWS = 4096
SEQ = 50
D = 128
NUM_WORKERS = 32
NSPLIT = 4
ROWS_PER_CALL = B_ROWS // NSPLIT
S_PER_W = ROWS_PER_CALL // NUM_WORKERS
GRP = 4
NGRP = S_PER_W // GRP
NBUF = 4


def _emb_body(idx_hbm, table_hbm, out_hbm, idx_v, rows, gsem, osem):
    wid = lax.axis_index("s") * 2 + lax.axis_index("c")
    base = wid * S_PER_W
    pltpu.sync_copy(idx_hbm.at[pl.ds(base, S_PER_W)], idx_v)

    def fire_group(g, b):
        for k in range(GRP):
            pltpu.async_copy(
                table_hbm.at[idx_v.at[g * GRP + k]], rows.at[b].at[k],
                gsem.at[b])

    for b in range(NBUF):
        fire_group(b, b)

    def body(i, carry):
        g0 = i * NBUF
        for b in range(NBUF):
            g = g0 + b
            for k in range(GRP):
                pltpu.make_async_copy(
                    table_hbm.at[idx_v.at[g * GRP + k]], rows.at[b].at[k],
                    gsem.at[b]).wait()
            pltpu.async_copy(
                rows.at[b], out_hbm.at[pl.ds(base + g * GRP, GRP)],
                osem.at[b])

            @pl.when(g + NBUF < NGRP)
            def _():
                pltpu.make_async_copy(
                    rows.at[b], out_hbm.at[pl.ds(base, GRP)],
                    osem.at[b]).wait()
                fire_group(g + NBUF, b)
        return carry

    lax.fori_loop(0, NGRP // NBUF, body, 0)

    for b in range(NBUF):
        pltpu.make_async_copy(
            rows.at[b], out_hbm.at[pl.ds(base, GRP)], osem.at[b]).wait()


def kernel(input, weight):
    idx = input.astype(jnp.int32)

    mesh = plsc.VectorSubcoreMesh(core_axis_name="c", subcore_axis_name="s")
    emb = functools.partial(
        pl.kernel,
        mesh=mesh,
        out_type=jax.ShapeDtypeStruct((ROWS_PER_CALL, SEQ, D), jnp.float32),
        scratch_types=[
            pltpu.VMEM((S_PER_W, SEQ), jnp.int32),
            pltpu.VMEM((NBUF, GRP, SEQ, D), jnp.float32),
            pltpu.SemaphoreType.DMA((NBUF,)),
            pltpu.SemaphoreType.DMA((NBUF,)),
        ],
    )(_emb_body)

    chunks = [
        emb(lax.slice_in_dim(idx, i * ROWS_PER_CALL, (i + 1) * ROWS_PER_CALL),
            weight)
        for i in range(NSPLIT)
    ]
    return jnp.concatenate(chunks, axis=0)


# speedup vs baseline: 1.0208x; 1.0208x over previous
import functools

import jax
import jax.numpy as jnp
from jax import lax
from jax.experimental import pallas as pl
from jax.experimental.pallas import tpu as pltpu
from jax.experimental.pallas import tpu_sc as plsc

B_ROWS = 4096
SEQ = 50
D = 128
NUM_WORKERS = 32
NSPLIT = 4
ROWS_PER_CALL = B_ROWS // NSPLIT
S_PER_W = ROWS_PER_CALL // NUM_WORKERS
GRP = 4
NGRP = S_PER_W // GRP
NBUF = 4


def _emb_body(idx_hbm, table_hbm, out_hbm, idx_v, rows, gsem, osem):
    wid = lax.axis_index("s") * 2 + lax.axis_index("c")
    base = wid * S_PER_W
    pltpu.sync_copy(idx_hbm.at[pl.ds(base, S_PER_W)], idx_v)

    def fire_group(g, b):
        for k in range(GRP):
            pltpu.async_copy(
                table_hbm.at[idx_v.at[g * GRP + k]], rows.at[b].at[k],
                gsem.at[b])

    for b in range(NBUF):
        fire_group(b, b)

    def body(i, carry):
        g0 = i * NBUF
        for b in range(NBUF):
            g = g0 + b
            for k in range(GRP):
                pltpu.make_async_copy(
                    table_hbm.at[idx_v.at[g * GRP + k]], rows.at[b].at[k],
                    gsem.at[b]).wait()
            pltpu.async_copy(
                rows.at[b], out_hbm.at[pl.ds(base + g * GRP, GRP)],
                osem.at[b])

            @pl.when(g + NBUF < NGRP)
            def _():
                pltpu.make_async_copy(
                    rows.at[b], out_hbm.at[pl.ds(base, GRP)],
                    osem.at[b]).wait()
                fire_group(g + NBUF, b)
        return carry

    lax.fori_loop(0, NGRP // NBUF, body, 0)

    for b in range(NBUF):
        pltpu.make_async_copy(
            rows.at[b], out_hbm.at[pl.ds(base, GRP)], osem.at[b]).wait()


def kernel(input, weight):
    idx = input.astype(jnp.int32)

    mesh = plsc.VectorSubcoreMesh(core_axis_name="c", subcore_axis_name="s")
    emb = functools.partial(
        pl.kernel,
        mesh=mesh,
        out_type=jax.ShapeDtypeStruct((ROWS_PER_CALL, SEQ, D), jnp.float32),
        scratch_types=[
            pltpu.VMEM((S_PER_W, SEQ), jnp.int32),
            pltpu.VMEM((NBUF, GRP, SEQ, D), jnp.float32),
            pltpu.SemaphoreType.DMA((NBUF,)),
            pltpu.SemaphoreType.DMA((NBUF,)),
        ],
    )(_emb_body)

    out = jnp.zeros((B_ROWS, SEQ, D), jnp.float32)
    for i in range(NSPLIT):
        chunk = emb(
            lax.slice_in_dim(idx, i * ROWS_PER_CALL, (i + 1) * ROWS_PER_CALL),
            weight)
        out = lax.dynamic_update_slice_in_dim(
            out, chunk, i * ROWS_PER_CALL, axis=0)
    return out


# speedup vs baseline: 1.8020x; 1.7654x over previous
import functools

import jax
import jax.numpy as jnp
from jax import lax
from jax.experimental import pallas as pl
from jax.experimental.pallas import tpu as pltpu
from jax.experimental.pallas import tpu_sc as plsc

B_ROWS = 4096
SEQ = 50
D = 128
NUM_WORKERS = 32
S_PER_W = B_ROWS // NUM_WORKERS
NBUF = 8


def _emb_body(idx_hbm, table_hbm, out_hbm, idx_v, rows, gsem, osem):
    wid = lax.axis_index("s") * 2 + lax.axis_index("c")
    base = wid * S_PER_W
    pltpu.sync_copy(idx_hbm.at[wid], idx_v)

    for b in range(NBUF):
        pltpu.async_copy(table_hbm.at[idx_v.at[b]], rows.at[b], gsem.at[b])

    def body(i, carry):
        j0 = i * NBUF
        for b in range(NBUF):
            j = j0 + b
            pltpu.make_async_copy(
                table_hbm.at[idx_v.at[j]], rows.at[b], gsem.at[b]).wait()
            pltpu.async_copy(rows.at[b], out_hbm.at[base + j], osem.at[b])

            @pl.when(j + NBUF < S_PER_W)
            def _():
                pltpu.make_async_copy(
                    rows.at[b], out_hbm.at[base + j], osem.at[b]).wait()
                pltpu.async_copy(
                    table_hbm.at[idx_v.at[j + NBUF]], rows.at[b], gsem.at[b])
        return carry

    lax.fori_loop(0, S_PER_W // NBUF, body, 0)

    for b in range(NBUF):
        pltpu.make_async_copy(
            rows.at[b], out_hbm.at[base], osem.at[b]).wait()


def kernel(input, weight):
    idx = input.reshape(NUM_WORKERS, S_PER_W, SEQ).astype(jnp.int32)

    mesh = plsc.VectorSubcoreMesh(core_axis_name="c", subcore_axis_name="s")
    emb = functools.partial(
        pl.kernel,
        mesh=mesh,
        out_type=jax.ShapeDtypeStruct((B_ROWS, SEQ, D), jnp.float32),
        scratch_types=[
            pltpu.VMEM((S_PER_W, SEQ), jnp.int32),
            pltpu.VMEM((NBUF, SEQ, D), jnp.float32),
            pltpu.SemaphoreType.DMA((NBUF,)),
            pltpu.SemaphoreType.DMA((NBUF,)),
        ],
    )(_emb_body)

    return emb(idx, weight)
